# Initial kernel scaffold; baseline (speedup 1.0000x reference)
import jax
import jax.numpy as jnp
from jax.experimental import pallas as pl


def kernel(x, pos, batch, diff, local_fea, shift_mlp, main_mlp, conv_mlp):
    raise NotImplementedError("write your pallas kernel here")


# speedup vs baseline: 4.1373x; 4.1373x over previous
import numpy as np

import jax
import jax.numpy as jnp
from jax import lax
from jax.experimental import pallas as pl

P = 128
PTN = 64
N = P * PTN
R2 = 0.3 ** 2
K = 9
NEG = -jnp.inf


def _new_points_np():
    xg = np.linspace(-0.2, 0.2, 4)
    yg = np.linspace(-0.2, 0.2, 4)
    xy = np.meshgrid(xg, yg)
    pts = np.array(xy).reshape(2, -1).T
    return np.concatenate([pts, np.zeros((pts.shape[0], 1))], axis=1).astype(np.float32)


def _plane_matrix_np():
    npts = _new_points_np()
    M = np.zeros((9, 48), dtype=np.float32)
    for j in range(16):
        for c in range(3):
            for d in range(3):
                M[3 * c + d, j * 3 + c] = npts[j, d]
    return M


def _main_kernel(pos_ref, lf_ref,
                 ws1l_ref, ws1p_ref, bs1_ref, ws2_ref, bs2_ref, ws3_ref, bs3_ref,
                 wc1_ref, bc1_ref, wc2_ref, bc2_ref,
                 wm1l_ref, wm1p_ref, bm1_ref, wm2_ref, bm2_ref, wm3_ref, bm3_ref,
                 rot9_ref, rotc_ref, posc_ref):
    pos = pos_ref[0]
    lf = lf_ref[0]

    lf_s = jnp.dot(lf, ws1l_ref[...], preferred_element_type=jnp.float32)
    h = jax.nn.relu(jnp.dot(pos, ws1p_ref[...], preferred_element_type=jnp.float32)
                    + lf_s + bs1_ref[...])
    h = jax.nn.relu(jnp.dot(h, ws2_ref[...], preferred_element_type=jnp.float32)
                    + bs2_ref[...])
    h = jax.nn.relu(jnp.dot(h, ws3_ref[...], preferred_element_type=jnp.float32)
                    + bs3_ref[...])
    n_pos = jnp.tanh(h) + pos

    gp = jnp.concatenate([pos, n_pos], axis=0)
    gpT = gp.T

    cols = [gp[:, c:c + 1] for c in range(3)]
    rows = [gpT[c:c + 1, :] for c in range(3)]
    d2 = ((cols[0] - rows[0]) ** 2 + (cols[1] - rows[1]) ** 2) \
        + (cols[2] - rows[2]) ** 2
    cur = jnp.where(d2 <= R2, -d2, NEG)

    iota = lax.broadcasted_iota(jnp.int32, (128, 128), 1)

    aggr = None
    for k in range(K):
        m = jnp.max(cur, axis=1, keepdims=True)
        ismax = cur == m
        idx = jnp.min(jnp.where(ismax, iota, 128), axis=1, keepdims=True)
        sel = iota == idx
        pj = [jnp.sum(jnp.where(sel, rows[c], 0.0), axis=1, keepdims=True)
              for c in range(3)]
        valid = m > NEG
        cur = jnp.where(sel, NEG, cur)
        feat = jnp.concatenate(
            [pj[0], pj[1], pj[2],
             pj[0] - cols[0], pj[1] - cols[1], pj[2] - cols[2]], axis=1)
        hmsg = jax.nn.relu(jnp.dot(feat, wc1_ref[...],
                                   preferred_element_type=jnp.float32) + bc1_ref[...])
        msg = jax.nn.relu(jnp.dot(hmsg, wc2_ref[...],
                                  preferred_element_type=jnp.float32) + bc2_ref[...])
        msg = jnp.where(valid, msg, NEG)
        aggr = msg if aggr is None else jnp.maximum(aggr, msg)

    lf_m = jnp.dot(lf, wm1l_ref[...], preferred_element_type=jnp.float32)
    h1 = jax.nn.relu(jnp.dot(aggr, wm1p_ref[...], preferred_element_type=jnp.float32)
                     + lf_m + bm1_ref[...])
    h2 = jax.nn.relu(jnp.dot(h1, wm2_ref[...], preferred_element_type=jnp.float32)
                     + bm2_ref[...])
    r9 = jax.nn.relu(jnp.dot(h2, wm3_ref[...], preferred_element_type=jnp.float32)
                     + bm3_ref[...])

    rc_cols = []
    for a in range(3):
        for b in range(3):
            rc_cols.append(r9[:, 0 + a:0 + a + 1] * r9[:, 0 + b:0 + b + 1]
                           + r9[:, 3 + a:3 + a + 1] * r9[:, 3 + b:3 + b + 1]
                           + r9[:, 6 + a:6 + a + 1] * r9[:, 6 + b:6 + b + 1])
    rc = jnp.concatenate(rc_cols, axis=1)

    rot9_ref[0] = r9[:PTN]
    rot9_ref[1] = r9[PTN:]
    rotc_ref[0] = rc[:PTN]
    rotc_ref[1] = rc[PTN:]
    posc_ref[0] = pos
    posc_ref[1] = n_pos


def _plane_kernel(rot9_ref, c48_ref, m_ref, out_ref):
    out_ref[0] = jnp.dot(rot9_ref[...], m_ref[...],
                         preferred_element_type=jnp.float32) + c48_ref[0]


def kernel(x, pos, batch, diff, local_fea, shift_mlp, main_mlp, conv_mlp):
    del x, batch, diff
    (ws1, bs1), (ws2, bs2), (ws3, bs3) = shift_mlp
    (wm1, bm1), (wm2, bm2), (wm3, bm3) = main_mlp
    (wc1, bc1), (wc2, bc2) = conv_mlp

    pos3 = pos.reshape(P, PTN, 3)
    lf3 = local_fea.reshape(P, 1, 768)

    args = [
        pos3, lf3,
        ws1[:, :768].T, ws1[:, 768:].T, bs1.reshape(1, -1),
        ws2.T, bs2.reshape(1, -1), ws3.T, bs3.reshape(1, -1),
        wc1.T, bc1.reshape(1, -1), wc2.T, bc2.reshape(1, -1),
        wm1[:, :768].T, wm1[:, 768:].T, bm1.reshape(1, -1),
        wm2.T, bm2.reshape(1, -1), wm3.T, bm3.reshape(1, -1),
    ]

    def fullspec(a):
        nd = a.ndim
        return pl.BlockSpec(a.shape, lambda p, _n=nd: (0,) * _n)

    in_specs = [
        pl.BlockSpec((1, PTN, 3), lambda p: (p, 0, 0)),
        pl.BlockSpec((1, 1, 768), lambda p: (p, 0, 0)),
    ] + [fullspec(a) for a in args[2:]]

    rot9, rotc, posc = pl.pallas_call(
        _main_kernel,
        grid=(P,),
        in_specs=in_specs,
        out_specs=[
            pl.BlockSpec((2, PTN, 9), lambda p: (0, p, 0)),
            pl.BlockSpec((2, PTN, 9), lambda p: (0, p, 0)),
            pl.BlockSpec((2, PTN, 3), lambda p: (0, p, 0)),
        ],
        out_shape=[
            jax.ShapeDtypeStruct((2, N, 9), jnp.float32),
            jax.ShapeDtypeStruct((2, N, 9), jnp.float32),
            jax.ShapeDtypeStruct((2, N, 3), jnp.float32),
        ],
    )(*args)

    rot9f = rot9.reshape(2 * N, 9)
    c48 = posc.reshape(8, 128, 48)
    M = jnp.asarray(_plane_matrix_np())

    plane = pl.pallas_call(
        _plane_kernel,
        grid=(P,),
        in_specs=[
            pl.BlockSpec((128, 9), lambda p: (p, 0)),
            pl.BlockSpec((1, 128, 48), lambda p: (p % 8, 0, 0)),
            pl.BlockSpec((9, 48), lambda p: (0, 0)),
        ],
        out_specs=pl.BlockSpec((1, 128, 48), lambda p: (p, 0, 0)),
        out_shape=jax.ShapeDtypeStruct((P, 128, 48), jnp.float32),
    )(rot9f, c48, M)

    plane_init = plane.reshape(P, 2048, 3)
    rot_constrain = rotc.reshape(2 * N, 3, 3)
    return plane_init, rot_constrain


# speedup vs baseline: 9.8588x; 2.3829x over previous
import numpy as np

import jax
import jax.numpy as jnp
from jax import lax
from jax.experimental import pallas as pl

P = 128
PTN = 64
N = P * PTN
R2 = 0.3 ** 2
K = 9
NEG = -jnp.inf
B = 8
G1 = P // B


def _new_points_np():
    xg = np.linspace(-0.2, 0.2, 4)
    yg = np.linspace(-0.2, 0.2, 4)
    xy = np.meshgrid(xg, yg)
    pts = np.array(xy).reshape(2, -1).T
    return np.concatenate([pts, np.zeros((pts.shape[0], 1))], axis=1).astype(np.float32)


def _plane_matrix_np():
    npts = _new_points_np()
    M = np.zeros((9, 48), dtype=np.float32)
    for j in range(16):
        for c in range(3):
            for d in range(3):
                M[3 * c + d, j * 3 + c] = npts[j, d]
    return M


def _dot(a, b):
    return jnp.dot(a, b, preferred_element_type=jnp.float32)


def _bdot(a, b):
    return lax.dot_general(a, b, (((2,), (1,)), ((0,), (0,))),
                           preferred_element_type=jnp.float32)


def _main_kernel(pos_ref, lf_ref,
                 ws1l_ref, ws1p_ref, bs1_ref, ws2_ref, bs2_ref, ws3_ref, bs3_ref,
                 wc1s_ref, wc1d_ref, bc1_ref, wc2_ref, bc2_ref,
                 wm1l_ref, wm1p_ref, bm1_ref, wm2_ref, bm2_ref, wm3_ref, bm3_ref,
                 rot9_ref, rotc_ref, posc_ref):
    pos2 = pos_ref[0]
    lf = lf_ref[0]

    lf_s = _dot(lf, ws1l_ref[...])
    lf_s3 = lax.broadcast_in_dim(lf_s, (B, PTN, 256), (0, 2)).reshape(B * PTN, 256)
    h = jax.nn.relu(_dot(pos2, ws1p_ref[...]) + lf_s3 + bs1_ref[...])
    h = jax.nn.relu(_dot(h, ws2_ref[...]) + bs2_ref[...])
    h = jax.nn.relu(_dot(h, ws3_ref[...]) + bs3_ref[...])
    npos2 = jnp.tanh(h) + pos2

    gp = jnp.concatenate([pos2.reshape(B, PTN, 3), npos2.reshape(B, PTN, 3)],
                         axis=1)

    Gm = lax.dot_general(gp, gp, (((2,), (2,)), ((0,), (0,))),
                         preferred_element_type=jnp.float32)
    iota_l = lax.broadcasted_iota(jnp.int32, (B, 128, 128), 2)
    iota_s = lax.broadcasted_iota(jnp.int32, (B, 128, 128), 1)
    eye = iota_l == iota_s
    nrm_r = jnp.sum(jnp.where(eye, Gm, 0.0), axis=1, keepdims=True)
    nrm_c = jnp.sum(gp * gp, axis=2, keepdims=True)
    d2 = nrm_c + nrm_r - 2.0 * Gm
    cur = jnp.where(d2 <= R2, -d2, NEG)

    gpA = _bdot(gp, lax.broadcast_in_dim(wc1s_ref[...], (B, 3, 64), (1, 2)))
    posiB = _bdot(gp, lax.broadcast_in_dim(wc1d_ref[...], (B, 3, 64), (1, 2)))
    bc1 = bc1_ref[...]
    wc2 = wc2_ref[...]
    bc2 = bc2_ref[...]

    aggr = None
    for k in range(K):
        m = jnp.max(cur, axis=2, keepdims=True)
        ismax = cur == m
        idx = jnp.min(jnp.where(ismax, iota_l, 128), axis=2, keepdims=True)
        sel = (iota_l == idx).astype(jnp.float32)
        valid = m > NEG
        cur = jnp.where(sel > 0, NEG, cur)
        hmsg = jax.nn.relu(_bdot(sel, gpA) - posiB + bc1)
        msg = jax.nn.relu(lax.dot_general(
            hmsg, wc2, (((2,), (0,)), ((), ())),
            preferred_element_type=jnp.float32) + bc2)
        msg = jnp.where(valid, msg, NEG)
        aggr = msg if aggr is None else jnp.maximum(aggr, msg)

    aggr2 = aggr.reshape(B * 128, 128)
    lf_m = _dot(lf, wm1l_ref[...])
    lf_m3 = lax.broadcast_in_dim(lf_m, (B, 128, 512), (0, 2)).reshape(B * 128, 512)
    h1 = jax.nn.relu(_dot(aggr2, wm1p_ref[...]) + lf_m3 + bm1_ref[...])
    h2 = jax.nn.relu(_dot(h1, wm2_ref[...]) + bm2_ref[...])
    r9 = jax.nn.relu(_dot(h2, wm3_ref[...]) + bm3_ref[...])

    rc_cols = []
    for a in range(3):
        for b in range(3):
            rc_cols.append(r9[:, 0 + a:0 + a + 1] * r9[:, 0 + b:0 + b + 1]
                           + r9[:, 3 + a:3 + a + 1] * r9[:, 3 + b:3 + b + 1]
                           + r9[:, 6 + a:6 + a + 1] * r9[:, 6 + b:6 + b + 1])
    rc = jnp.concatenate(rc_cols, axis=1)

    r93 = r9.reshape(B, 128, 9)
    rc3 = rc.reshape(B, 128, 9)
    rot9_ref[0] = r93[:, :PTN].reshape(B * PTN, 9)
    rot9_ref[1] = r93[:, PTN:].reshape(B * PTN, 9)
    rotc_ref[0] = rc3[:, :PTN].reshape(B * PTN, 9)
    rotc_ref[1] = rc3[:, PTN:].reshape(B * PTN, 9)
    posc_ref[0] = pos2
    posc_ref[1] = npos2


def _plane_kernel(rot9_ref, c48_ref, m_ref, out_ref):
    out_ref[0] = _dot(rot9_ref[...], m_ref[...]) + c48_ref[...]


def kernel(x, pos, batch, diff, local_fea, shift_mlp, main_mlp, conv_mlp):
    del x, batch, diff
    (ws1, bs1), (ws2, bs2), (ws3, bs3) = shift_mlp
    (wm1, bm1), (wm2, bm2), (wm3, bm3) = main_mlp
    (wc1, bc1), (wc2, bc2) = conv_mlp

    pos3 = pos.reshape(G1, B * PTN, 3)
    lf3 = local_fea.reshape(G1, B, 768)

    args = [
        pos3, lf3,
        ws1[:, :768].T, ws1[:, 768:].T, bs1.reshape(1, -1),
        ws2.T, bs2.reshape(1, -1), ws3.T, bs3.reshape(1, -1),
        wc1[:, :3].T + wc1[:, 3:].T, wc1[:, 3:].T, bc1.reshape(1, 1, -1),
        wc2.T, bc2.reshape(1, 1, -1),
        wm1[:, :768].T, wm1[:, 768:].T, bm1.reshape(1, -1),
        wm2.T, bm2.reshape(1, -1), wm3.T, bm3.reshape(1, -1),
    ]

    def fullspec(a):
        nd = a.ndim
        return pl.BlockSpec(a.shape, lambda p, _n=nd: (0,) * _n)

    in_specs = [
        pl.BlockSpec((1, B * PTN, 3), lambda p: (p, 0, 0)),
        pl.BlockSpec((1, B, 768), lambda p: (p, 0, 0)),
    ] + [fullspec(a) for a in args[2:]]

    rot9, rotc, posc = pl.pallas_call(
        _main_kernel,
        grid=(G1,),
        in_specs=in_specs,
        out_specs=[
            pl.BlockSpec((2, B * PTN, 9), lambda p: (0, p, 0)),
            pl.BlockSpec((2, B * PTN, 9), lambda p: (0, p, 0)),
            pl.BlockSpec((2, B * PTN, 3), lambda p: (0, p, 0)),
        ],
        out_shape=[
            jax.ShapeDtypeStruct((2, N, 9), jnp.float32),
            jax.ShapeDtypeStruct((2, N, 9), jnp.float32),
            jax.ShapeDtypeStruct((2, N, 3), jnp.float32),
        ],
    )(*args)

    rot9f = rot9.reshape(2 * N, 9)
    c48 = posc.reshape(1024, 48)
    M = jnp.asarray(_plane_matrix_np())

    plane = pl.pallas_call(
        _plane_kernel,
        grid=(16,),
        in_specs=[
            pl.BlockSpec((1024, 9), lambda p: (p, 0)),
            pl.BlockSpec((1024, 48), lambda p: (0, 0)),
            pl.BlockSpec((9, 48), lambda p: (0, 0)),
        ],
        out_specs=pl.BlockSpec((1, 1024, 48), lambda p: (p, 0, 0)),
        out_shape=jax.ShapeDtypeStruct((16, 1024, 48), jnp.float32),
    )(rot9f, c48, M)

    plane_init = plane.reshape(P, 2048, 3)
    rot_constrain = rotc.reshape(2 * N, 3, 3)
    return plane_init, rot_constrain


# speedup vs baseline: 10.9901x; 1.1147x over previous
import numpy as np

import jax
import jax.numpy as jnp
from jax import lax
from jax.experimental import pallas as pl

P = 128
PTN = 64
N = P * PTN
R2 = 0.3 ** 2
K = 9
NEG = -jnp.inf
B = 8
G1 = P // B


def _new_points_np():
    xg = np.linspace(-0.2, 0.2, 4)
    yg = np.linspace(-0.2, 0.2, 4)
    xy = np.meshgrid(xg, yg)
    pts = np.array(xy).reshape(2, -1).T
    return np.concatenate([pts, np.zeros((pts.shape[0], 1))], axis=1).astype(np.float32)


def _plane_matrix_np():
    npts = _new_points_np()
    M = np.zeros((9, 48), dtype=np.float32)
    for j in range(16):
        for c in range(3):
            for d in range(3):
                M[3 * c + d, j * 3 + c] = npts[j, d]
    return M


def _dot(a, b):
    return jnp.dot(a, b, preferred_element_type=jnp.float32)


def _bdot(a, b):
    return lax.dot_general(a, b, (((2,), (1,)), ((0,), (0,))),
                           preferred_element_type=jnp.float32)


def _main_kernel(pos_ref, lf_ref,
                 ws1l_ref, ws1p_ref, bs1_ref, ws2_ref, bs2_ref, ws3_ref, bs3_ref,
                 wc1s_ref, wc1d_ref, bc1_ref, wc2_ref, bc2_ref,
                 wm1l_ref, wm1p_ref, bm1_ref, wm2_ref, bm2_ref, wm3_ref, bm3_ref,
                 rot9_ref, rotc_ref, posc_ref):
    pos2 = pos_ref[0]
    lf = lf_ref[0]

    lf_s = _dot(lf, ws1l_ref[...])
    lf_s3 = lax.broadcast_in_dim(lf_s, (B, PTN, 256), (0, 2)).reshape(B * PTN, 256)
    h = jax.nn.relu(_dot(pos2, ws1p_ref[...]) + lf_s3 + bs1_ref[...])
    h = jax.nn.relu(_dot(h, ws2_ref[...]) + bs2_ref[...])
    h = jax.nn.relu(_dot(h, ws3_ref[...]) + bs3_ref[...])
    npos2 = jnp.tanh(h) + pos2

    gp = jnp.concatenate([pos2.reshape(B, PTN, 3), npos2.reshape(B, PTN, 3)],
                         axis=1)

    Gm = lax.dot_general(gp, gp, (((2,), (2,)), ((0,), (0,))),
                         preferred_element_type=jnp.float32)
    iota_l = lax.broadcasted_iota(jnp.int32, (B, 128, 128), 2)
    iota_s = lax.broadcasted_iota(jnp.int32, (B, 128, 128), 1)
    eye = iota_l == iota_s
    nrm_r = jnp.sum(jnp.where(eye, Gm, 0.0), axis=1, keepdims=True)
    nrm_c = jnp.sum(gp * gp, axis=2, keepdims=True)
    d2 = nrm_c + nrm_r - 2.0 * Gm
    kbits = lax.bitcast_convert_type(jnp.maximum(d2, 0.0), jnp.int32)
    kbits = jnp.bitwise_or(jnp.bitwise_and(kbits, -128), iota_l)
    key = lax.bitcast_convert_type(kbits, jnp.float32)
    cur = jnp.where(d2 <= R2, key, jnp.inf)

    gpA = _bdot(gp, lax.broadcast_in_dim(wc1s_ref[...], (B, 3, 64), (1, 2)))
    posiB = _bdot(gp, lax.broadcast_in_dim(wc1d_ref[...], (B, 3, 64), (1, 2)))
    bc1 = bc1_ref[...]
    wc2 = wc2_ref[...]
    bc2 = bc2_ref[...]

    aggr = None
    for k in range(K):
        m = jnp.min(cur, axis=2, keepdims=True)
        selb = cur == m
        sel = selb.astype(jnp.float32)
        valid = m < jnp.inf
        cur = jnp.where(selb, jnp.inf, cur)
        hmsg = jax.nn.relu(_bdot(sel, gpA) - posiB + bc1)
        msg = jax.nn.relu(lax.dot_general(
            hmsg, wc2, (((2,), (0,)), ((), ())),
            preferred_element_type=jnp.float32) + bc2)
        msg = jnp.where(valid, msg, NEG)
        aggr = msg if aggr is None else jnp.maximum(aggr, msg)

    aggr2 = aggr.reshape(B * 128, 128)
    lf_m = _dot(lf, wm1l_ref[...])
    lf_m3 = lax.broadcast_in_dim(lf_m, (B, 128, 512), (0, 2)).reshape(B * 128, 512)
    h1 = jax.nn.relu(_dot(aggr2, wm1p_ref[...]) + lf_m3 + bm1_ref[...])
    h2 = jax.nn.relu(_dot(h1, wm2_ref[...]) + bm2_ref[...])
    r9 = jax.nn.relu(_dot(h2, wm3_ref[...]) + bm3_ref[...])

    rc_cols = []
    for a in range(3):
        for b in range(3):
            rc_cols.append(r9[:, 0 + a:0 + a + 1] * r9[:, 0 + b:0 + b + 1]
                           + r9[:, 3 + a:3 + a + 1] * r9[:, 3 + b:3 + b + 1]
                           + r9[:, 6 + a:6 + a + 1] * r9[:, 6 + b:6 + b + 1])
    rc = jnp.concatenate(rc_cols, axis=1)

    r93 = r9.reshape(B, 128, 9)
    rc3 = rc.reshape(B, 128, 9)
    rot9_ref[0] = r93[:, :PTN].reshape(B * PTN, 9)
    rot9_ref[1] = r93[:, PTN:].reshape(B * PTN, 9)
    rotc_ref[0] = rc3[:, :PTN].reshape(B * PTN, 9)
    rotc_ref[1] = rc3[:, PTN:].reshape(B * PTN, 9)
    posc_ref[0] = pos2
    posc_ref[1] = npos2


def _plane_kernel(rot9_ref, c48_ref, m_ref, out_ref):
    out_ref[0] = _dot(rot9_ref[...], m_ref[...]) + c48_ref[...]


def kernel(x, pos, batch, diff, local_fea, shift_mlp, main_mlp, conv_mlp):
    del x, batch, diff
    (ws1, bs1), (ws2, bs2), (ws3, bs3) = shift_mlp
    (wm1, bm1), (wm2, bm2), (wm3, bm3) = main_mlp
    (wc1, bc1), (wc2, bc2) = conv_mlp

    pos3 = pos.reshape(G1, B * PTN, 3)
    lf3 = local_fea.reshape(G1, B, 768)

    args = [
        pos3, lf3,
        ws1[:, :768].T, ws1[:, 768:].T, bs1.reshape(1, -1),
        ws2.T, bs2.reshape(1, -1), ws3.T, bs3.reshape(1, -1),
        wc1[:, :3].T + wc1[:, 3:].T, wc1[:, 3:].T, bc1.reshape(1, 1, -1),
        wc2.T, bc2.reshape(1, 1, -1),
        wm1[:, :768].T, wm1[:, 768:].T, bm1.reshape(1, -1),
        wm2.T, bm2.reshape(1, -1), wm3.T, bm3.reshape(1, -1),
    ]

    def fullspec(a):
        nd = a.ndim
        return pl.BlockSpec(a.shape, lambda p, _n=nd: (0,) * _n)

    in_specs = [
        pl.BlockSpec((1, B * PTN, 3), lambda p: (p, 0, 0)),
        pl.BlockSpec((1, B, 768), lambda p: (p, 0, 0)),
    ] + [fullspec(a) for a in args[2:]]

    rot9, rotc, posc = pl.pallas_call(
        _main_kernel,
        grid=(G1,),
        in_specs=in_specs,
        out_specs=[
            pl.BlockSpec((2, B * PTN, 9), lambda p: (0, p, 0)),
            pl.BlockSpec((2, B * PTN, 9), lambda p: (0, p, 0)),
            pl.BlockSpec((2, B * PTN, 3), lambda p: (0, p, 0)),
        ],
        out_shape=[
            jax.ShapeDtypeStruct((2, N, 9), jnp.float32),
            jax.ShapeDtypeStruct((2, N, 9), jnp.float32),
            jax.ShapeDtypeStruct((2, N, 3), jnp.float32),
        ],
    )(*args)

    rot9f = rot9.reshape(2 * N, 9)
    c48 = posc.reshape(1024, 48)
    M = jnp.asarray(_plane_matrix_np())

    plane = pl.pallas_call(
        _plane_kernel,
        grid=(16,),
        in_specs=[
            pl.BlockSpec((1024, 9), lambda p: (p, 0)),
            pl.BlockSpec((1024, 48), lambda p: (0, 0)),
            pl.BlockSpec((9, 48), lambda p: (0, 0)),
        ],
        out_specs=pl.BlockSpec((1, 1024, 48), lambda p: (p, 0, 0)),
        out_shape=jax.ShapeDtypeStruct((16, 1024, 48), jnp.float32),
    )(rot9f, c48, M)

    plane_init = plane.reshape(P, 2048, 3)
    rot_constrain = rotc.reshape(2 * N, 3, 3)
    return plane_init, rot_constrain
